# baseline (speedup 1.0000x reference)
import jax
import jax.numpy as jnp
from jax import lax
from jax.experimental import pallas as pl
from jax.experimental.pallas import tpu as pltpu
from jax.experimental.pallas import tpu_sc as plsc

NC = 2
NS = 16
NW = NC * NS

EMB = 32
BATCH = 4096
HIST = 200
LANE = 128
NTC = (BATCH // LANE) * HIST
NTC_PER_W = NTC // NW
NB = 2
TC_PER_G = 5
NGROUPS = NTC_PER_W // TC_PER_G
G_ROWS = TC_PER_G * LANE
TILE_ROWS = BATCH // LANE
NJT = EMB // 8
OUT_TROWS = HIST * NJT * TILE_ROWS


def _emb_lookup(idx, weight):
    mesh = plsc.VectorSubcoreMesh(
        core_axis_name="c", subcore_axis_name="s", num_cores=NC, num_subcores=NS
    )

    def body(idx_hbm, table_hbm, out_hbm, idx_v, rows_v, t_v, gsems, osems, isems):
        wid = lax.axis_index("s") * NC + lax.axis_index("c")
        lanes = jax.lax.iota(jnp.int32, 16)

        @pl.loop(0, NGROUPS, step=NB)
        def _outer(g0):
            for b in range(NB):
                @pl.when(g0 > 0)
                def _wait_prev_store():
                    pltpu.make_async_copy(
                        out_hbm.at[pl.ds(0, TC_PER_G * NJT)], t_v.at[b], osems[b]
                    ).wait()
                tc0 = wid * NTC_PER_W + (g0 + b) * TC_PER_G
                pltpu.async_copy(
                    idx_hbm.at[pl.ds(tc0, TC_PER_G)], idx_v.at[b], isems[b]
                )
            for b in range(NB):
                pltpu.make_async_copy(
                    idx_hbm.at[pl.ds(0, TC_PER_G)], idx_v.at[b], isems[b]
                ).wait()
                for j in range(TC_PER_G):
                    pltpu.async_copy(
                        table_hbm.at[idx_v.at[b].at[j]],
                        rows_v.at[b].at[pl.ds(j * LANE, LANE)],
                        gsems[b],
                    )
            for b in range(NB):
                pltpu.make_async_copy(
                    table_hbm.at[pl.ds(0, G_ROWS)], rows_v.at[b], gsems[b]
                ).wait()

                @pl.loop(0, TC_PER_G * EMB)
                def _transpose(u):
                    tl = u // EMB
                    j = u % EMB
                    trow = tl * NJT + j // 8
                    jr = j % 8
                    col = jnp.full((16,), j, dtype=jnp.int32)
                    for k in range(8):
                        rows = tl * LANE + k * 16 + lanes
                        vals = plsc.load_gather(rows_v.at[b], [rows, col])
                        t_v.at[b].at[trow].at[jr][pl.ds(k * 16, 16)] = vals

                tc0 = wid * NTC_PER_W + (g0 + b) * TC_PER_G
                for tl in range(TC_PER_G):
                    tc = tc0 + tl
                    h = tc // TILE_ROWS
                    bt = tc % TILE_ROWS
                    for jt in range(NJT):
                        pltpu.async_copy(
                            t_v.at[b].at[tl * NJT + jt],
                            out_hbm.at[(h * NJT + jt) * TILE_ROWS + bt],
                            osems[b],
                        )

        for b in range(NB):
            pltpu.make_async_copy(
                out_hbm.at[pl.ds(0, TC_PER_G * NJT)], t_v.at[b], osems[b]
            ).wait()

    run = pl.kernel(
        body,
        out_type=jax.ShapeDtypeStruct((OUT_TROWS, 8, LANE), jnp.float32),
        mesh=mesh,
        scratch_types=[
            pltpu.VMEM((NB, TC_PER_G, LANE), jnp.int32),
            pltpu.VMEM((NB, G_ROWS, EMB), jnp.float32),
            pltpu.VMEM((NB, TC_PER_G * NJT, 8, LANE), jnp.float32),
            [pltpu.SemaphoreType.DMA] * NB,
            [pltpu.SemaphoreType.DMA] * NB,
            [pltpu.SemaphoreType.DMA] * NB,
        ],
        compiler_params=pltpu.CompilerParams(
            use_tc_tiling_on_sc=False, needs_layout_passes=False
        ),
    )
    return run(idx, weight)


def kernel(x, weight):
    idx = x.T.astype(jnp.int32).reshape(NTC, LANE)
    o = _emb_lookup(idx, weight)
    o5 = o.reshape(HIST, NJT, TILE_ROWS, 8, LANE)
    return o5.transpose(2, 4, 0, 1, 3).reshape(BATCH, HIST, EMB)


# speedup vs baseline: 1.1088x; 1.1088x over previous
import jax
import jax.numpy as jnp
from jax import lax
from jax.experimental import pallas as pl
from jax.experimental.pallas import tpu as pltpu
from jax.experimental.pallas import tpu_sc as plsc

NC = 2
NS = 16
NW = NC * NS

EMB = 32
BATCH = 4096
HIST = 200
LANE = 128
NTC = (BATCH // LANE) * HIST
NTC_PER_W = NTC // NW
NB = 2
TC_PER_G = 5
NGROUPS = NTC_PER_W // TC_PER_G
G_ROWS = TC_PER_G * LANE
TILE_ROWS = BATCH // LANE
NJT = EMB // 8
TILE_F = 8 * LANE
T_FLAT = TC_PER_G * NJT * TILE_F
OUT_F = HIST * NJT * TILE_ROWS * TILE_F


def _emb_lookup(idx, weight):
    mesh = plsc.VectorSubcoreMesh(
        core_axis_name="c", subcore_axis_name="s", num_cores=NC, num_subcores=NS
    )

    def body(idx_hbm, table_hbm, out_hbm, idx_v, rows_v, t_v, gsems, osems, isems):
        wid = lax.axis_index("s") * NC + lax.axis_index("c")
        lanes = jax.lax.iota(jnp.int32, 16)
        idx_half = [(jnp.int32(h * 16) + lanes) * LANE for h in (0, 1)]

        @pl.loop(0, NGROUPS, step=NB)
        def _outer(g0):
            for b in range(NB):
                @pl.when(g0 > 0)
                def _wait_prev_store():
                    pltpu.make_async_copy(
                        out_hbm.at[pl.ds(0, T_FLAT)], t_v.at[b], osems[b]
                    ).wait()
                tc0 = wid * NTC_PER_W + (g0 + b) * TC_PER_G
                pltpu.async_copy(
                    idx_hbm.at[pl.ds(tc0, TC_PER_G)], idx_v.at[b], isems[b]
                )
            for b in range(NB):
                pltpu.make_async_copy(
                    idx_hbm.at[pl.ds(0, TC_PER_G)], idx_v.at[b], isems[b]
                ).wait()
                for j in range(TC_PER_G):
                    pltpu.async_copy(
                        table_hbm.at[idx_v.at[b].at[j]],
                        rows_v.at[b].at[pl.ds(j * LANE, LANE)],
                        gsems[b],
                    )
            for b in range(NB):
                pltpu.make_async_copy(
                    table_hbm.at[pl.ds(0, G_ROWS)], rows_v.at[b], gsems[b]
                ).wait()

                @pl.loop(0, LANE)
                def _transpose(bc):
                    for tl in range(TC_PER_G):
                        row = tl * LANE + bc
                        tbase = tl * (NJT * TILE_F) + bc
                        for h in range(2):
                            vals = rows_v.at[b].at[row][pl.ds(h * 16, 16)]
                            plsc.store_scatter(
                                t_v.at[b], [idx_half[h] + tbase], vals
                            )

                tc0 = wid * NTC_PER_W + (g0 + b) * TC_PER_G
                for tl in range(TC_PER_G):
                    tc = tc0 + tl
                    hh = tc // TILE_ROWS
                    bt = tc % TILE_ROWS
                    for jt in range(NJT):
                        pltpu.async_copy(
                            t_v.at[b].at[pl.ds((tl * NJT + jt) * TILE_F, TILE_F)],
                            out_hbm.at[
                                pl.ds(((hh * NJT + jt) * TILE_ROWS + bt) * TILE_F,
                                      TILE_F)
                            ],
                            osems[b],
                        )

        for b in range(NB):
            pltpu.make_async_copy(
                out_hbm.at[pl.ds(0, T_FLAT)], t_v.at[b], osems[b]
            ).wait()

    run = pl.kernel(
        body,
        out_type=jax.ShapeDtypeStruct((OUT_F,), jnp.float32),
        mesh=mesh,
        scratch_types=[
            pltpu.VMEM((NB, TC_PER_G, LANE), jnp.int32),
            pltpu.VMEM((NB, G_ROWS, EMB), jnp.float32),
            pltpu.VMEM((NB, T_FLAT), jnp.float32),
            [pltpu.SemaphoreType.DMA] * NB,
            [pltpu.SemaphoreType.DMA] * NB,
            [pltpu.SemaphoreType.DMA] * NB,
        ],
        compiler_params=pltpu.CompilerParams(
            use_tc_tiling_on_sc=False, needs_layout_passes=False
        ),
    )
    return run(idx, weight)


def kernel(x, weight):
    idx = x.T.astype(jnp.int32).reshape(NTC, LANE)
    o = _emb_lookup(idx, weight)
    o5 = o.reshape(HIST, NJT, TILE_ROWS, 8, LANE)
    return o5.transpose(2, 4, 0, 1, 3).reshape(BATCH, HIST, EMB)


# speedup vs baseline: 1.2854x; 1.1593x over previous
import jax
import jax.numpy as jnp
from jax import lax
from jax.experimental import pallas as pl
from jax.experimental.pallas import tpu as pltpu
from jax.experimental.pallas import tpu_sc as plsc

NC = 2
NS = 16
NW = NC * NS

EMB = 32
BATCH = 4096
HIST = 200
LANE = 128
NTC = (BATCH // LANE) * HIST
NTC_PER_W = NTC // NW
NB = 2
TC_PER_G = 5
NGROUPS = NTC_PER_W // TC_PER_G
G_ROWS = TC_PER_G * LANE
TILE_ROWS = BATCH // LANE
NJT = EMB // 8
TILE_F = 8 * LANE
T_FLAT = TC_PER_G * NJT * TILE_F
OUT_F = HIST * NJT * TILE_ROWS * TILE_F


def _emb_lookup(idx, weight):
    mesh = plsc.VectorSubcoreMesh(
        core_axis_name="c", subcore_axis_name="s", num_cores=NC, num_subcores=NS
    )

    def body(idx_hbm, table_hbm, out_hbm, idx_v, rows_v, t_v, gsems, osems, isems):
        wid = lax.axis_index("s") * NC + lax.axis_index("c")
        lanes = jax.lax.iota(jnp.int32, 16)
        idx_half = [(jnp.int32(h * 16) + lanes) * LANE for h in (0, 1)]

        @pl.loop(0, NGROUPS, step=NB)
        def _outer(g0):
            for b in range(NB):
                @pl.when(g0 > 0)
                def _wait_prev_store():
                    pltpu.make_async_copy(
                        out_hbm.at[pl.ds(0, T_FLAT)], t_v.at[b], osems[b]
                    ).wait()
                tc0 = wid * NTC_PER_W + (g0 + b) * TC_PER_G
                pltpu.async_copy(
                    idx_hbm.at[pl.ds(tc0, TC_PER_G)], idx_v.at[b], isems[b]
                )
            for b in range(NB):
                pltpu.make_async_copy(
                    idx_hbm.at[pl.ds(0, TC_PER_G)], idx_v.at[b], isems[b]
                ).wait()
                for j in range(TC_PER_G):
                    pltpu.async_copy(
                        table_hbm.at[idx_v.at[b].at[j]],
                        rows_v.at[b].at[pl.ds(j * LANE, LANE)],
                        gsems[b],
                    )
            for b in range(NB):
                pltpu.make_async_copy(
                    table_hbm.at[pl.ds(0, G_ROWS)], rows_v.at[b], gsems[b]
                ).wait()

                @plsc.parallel_loop(0, LANE, unroll=4)
                def _transpose(bc):
                    for tl in range(TC_PER_G):
                        row = tl * LANE + bc
                        tbase = tl * (NJT * TILE_F) + bc
                        for h in range(2):
                            vals = rows_v.at[b].at[row][pl.ds(h * 16, 16)]
                            plsc.store_scatter(
                                t_v.at[b], [idx_half[h] + tbase], vals
                            )

                tc0 = wid * NTC_PER_W + (g0 + b) * TC_PER_G
                for tl in range(TC_PER_G):
                    tc = tc0 + tl
                    hh = tc // TILE_ROWS
                    bt = tc % TILE_ROWS
                    for jt in range(NJT):
                        pltpu.async_copy(
                            t_v.at[b].at[pl.ds((tl * NJT + jt) * TILE_F, TILE_F)],
                            out_hbm.at[
                                pl.ds(((hh * NJT + jt) * TILE_ROWS + bt) * TILE_F,
                                      TILE_F)
                            ],
                            osems[b],
                        )

        for b in range(NB):
            pltpu.make_async_copy(
                out_hbm.at[pl.ds(0, T_FLAT)], t_v.at[b], osems[b]
            ).wait()

    run = pl.kernel(
        body,
        out_type=jax.ShapeDtypeStruct((OUT_F,), jnp.float32),
        mesh=mesh,
        scratch_types=[
            pltpu.VMEM((NB, TC_PER_G, LANE), jnp.int32),
            pltpu.VMEM((NB, G_ROWS, EMB), jnp.float32),
            pltpu.VMEM((NB, T_FLAT), jnp.float32),
            [pltpu.SemaphoreType.DMA] * NB,
            [pltpu.SemaphoreType.DMA] * NB,
            [pltpu.SemaphoreType.DMA] * NB,
        ],
        compiler_params=pltpu.CompilerParams(
            use_tc_tiling_on_sc=False, needs_layout_passes=False
        ),
    )
    return run(idx, weight)


def kernel(x, weight):
    idx = x.T.astype(jnp.int32).reshape(NTC, LANE)
    o = _emb_lookup(idx, weight)
    o5 = o.reshape(HIST, NJT, TILE_ROWS, 8, LANE)
    return o5.transpose(2, 4, 0, 1, 3).reshape(BATCH, HIST, EMB)
